# baseline (speedup 1.0000x reference)
import jax
import jax.numpy as jnp
from jax import lax
from jax.experimental import pallas as pl

N_OBJ = 128
T = 100
D = 64
ROWS = 128


def _concat_body(ids_ref, s_ref, tab_ref, o_ref):
    ids = ids_ref[...]
    cols = lax.broadcasted_iota(jnp.int32, (N_OBJ, N_OBJ), 1)
    onehot = (ids == cols).astype(jnp.float32)
    emb = jnp.dot(onehot, tab_ref[...],
                  preferred_element_type=jnp.float32)
    embb = jnp.broadcast_to(emb[None, :, None, :], (1, N_OBJ, T, D))
    o_ref[...] = jnp.concatenate([s_ref[...], embb], axis=-1)


def kernel(states, table, obj_ids):
    B, N, t, d = states.shape
    ids2d = obj_ids.reshape(N, 1)
    return pl.pallas_call(
        _concat_body,
        grid=(B,),
        in_specs=[
            pl.BlockSpec((N, 1), lambda g: (0, 0)),
            pl.BlockSpec((1, N, t, d), lambda g: (g, 0, 0, 0)),
            pl.BlockSpec((N, d), lambda g: (0, 0)),
        ],
        out_specs=pl.BlockSpec((1, N, t, 2 * d), lambda g: (g, 0, 0, 0)),
        out_shape=jax.ShapeDtypeStruct((B, N, t, 2 * d), jnp.float32),
    )(ids2d, states, table)
